# baseline (speedup 1.0000x reference)
import functools

import jax
import jax.numpy as jnp
from jax import lax
from jax.experimental import pallas as pl
from jax.experimental.pallas import tpu as pltpu
from jax.experimental.pallas import tpu_sc as plsc

N_NODES = 10000
N_EDGES = 320000
D = 128

NC = 2
NS = 16
L = 16

CHUNK = 128
NCH = 80
N_CHUNKS = NC * NS * NCH
E_PAD = N_CHUNKS * CHUNK

ROWS_PER_TILE = 624
REM_BASE = ROWS_PER_TILE * NS
REM_ROWS = N_NODES - REM_BASE

_mesh = plsc.VectorSubcoreMesh(core_axis_name="c", subcore_axis_name="s")


@functools.partial(
    pl.kernel,
    out_type=jax.ShapeDtypeStruct((NC, N_NODES, D), jnp.float32),
    mesh=_mesh,
    compiler_params=pltpu.CompilerParams(needs_layout_passes=False),
    scratch_types=[
        pltpu.VMEM_SHARED((N_NODES, D), jnp.float32),
        pltpu.VMEM((3, CHUNK), jnp.int32),
        pltpu.VMEM((3, CHUNK), jnp.int32),
        pltpu.VMEM((3, CHUNK), jnp.int32),
        pltpu.VMEM((CHUNK, D), jnp.float32),
        pltpu.VMEM((CHUNK, D), jnp.float32),
        pltpu.VMEM((CHUNK, D), jnp.float32),
        pltpu.SemaphoreType.DMA,
        pltpu.SemaphoreType.DMA,
        pltpu.SemaphoreType.DMA,
        pltpu.SemaphoreType.DMA,
        pltpu.SemaphoreType.DMA,
        pltpu.SemaphoreType.DMA,
        pltpu.SemaphoreType.DMA,
        pltpu.SemaphoreType.DMA,
        pltpu.SemaphoreType.DMA,
    ],
)
def _sc_agg(h_hbm, packed_hbm, zeros_hbm, out_hbm,
            acc, ib0, ib1, ib2, rb0, rb1, rb2,
            si0, si1, si2, sg0, sg1, sg2, ss0, ss1, ss2):
    cid = lax.axis_index("c")
    sid = lax.axis_index("s")
    wid = sid * NC + cid
    first = wid * NCH

    ibs = (ib0, ib1, ib2)
    rbs = (rb0, rb1, rb2)
    sis = (si0, si1, si2)
    sgs = (sg0, sg1, sg2)
    sss = (ss0, ss1, ss2)

    pltpu.sync_copy(zeros_hbm.at[pl.ds(sid * ROWS_PER_TILE, ROWS_PER_TILE)],
                    acc.at[pl.ds(sid * ROWS_PER_TILE, ROWS_PER_TILE)])

    @pl.when(sid == NS - 1)
    def _zero_rem():
        pltpu.sync_copy(zeros_hbm.at[pl.ds(REM_BASE, REM_ROWS)],
                        acc.at[pl.ds(REM_BASE, REM_ROWS)])

    plsc.subcore_barrier()

    def idx_start(c, p):
        pltpu.async_copy(packed_hbm.at[first + c], ibs[p], sis[p])

    def idx_wait(p):
        pltpu.make_async_copy(packed_hbm.at[first], ibs[p], sis[p]).wait()

    def gather_start(p):
        pltpu.async_copy(h_hbm.at[ibs[p].at[0]], rbs[p], sgs[p])

    def gather_wait(p):
        pltpu.make_async_copy(h_hbm.at[ibs[p].at[0]], rbs[p], sgs[p]).wait()

    def scatter_start(p):
        pltpu.async_copy(rbs[p], acc.at[ibs[p].at[1]], sss[p], add=True)

    def scatter_wait(p):
        pltpu.make_async_copy(rbs[p], acc.at[ibs[p].at[1]], sss[p]).wait()

    two = jnp.broadcast_to(jnp.int32(2), (L,))

    def scale(p):
        def body(e, c2):
            wbits = plsc.load_gather(ibs[p], [two, jnp.broadcast_to(e, (L,))])
            w16 = plsc.bitcast(wbits, jnp.float32)
            for j in range(D // L):
                sl = pl.ds(j * L, L)
                rbs[p][e, sl] = rbs[p][e, sl] * w16
            return c2

        lax.fori_loop(0, CHUNK, body, 0, unroll=2)

    def chunk_body(g, carry):
        idx_start(g, 0)
        idx_wait(0)
        gather_start(0)
        gather_wait(0)
        scale(0)
        scatter_start(0)
        scatter_wait(0)
        return carry

    lax.fori_loop(0, NCH, chunk_body, 0)

    plsc.subcore_barrier()

    pltpu.sync_copy(acc.at[pl.ds(sid * ROWS_PER_TILE, ROWS_PER_TILE)],
                    out_hbm.at[cid, pl.ds(sid * ROWS_PER_TILE, ROWS_PER_TILE)])

    @pl.when(sid == NS - 1)
    def _write_rem():
        pltpu.sync_copy(acc.at[pl.ds(REM_BASE, REM_ROWS)],
                        out_hbm.at[cid, pl.ds(REM_BASE, REM_ROWS)])


_BLK = 1000


def _tc_body(p_ref, h_ref, wrel_ref, wroot_ref, b_ref, o_ref):
    agg = p_ref[0] + p_ref[1]
    o_ref[...] = (
        jnp.dot(agg, wrel_ref[...], preferred_element_type=jnp.float32)
        + jnp.dot(h_ref[...], wroot_ref[...], preferred_element_type=jnp.float32)
        + b_ref[...]
    )


_tc_combine = pl.pallas_call(
    _tc_body,
    grid=(N_NODES // _BLK,),
    in_specs=[
        pl.BlockSpec((NC, _BLK, D), lambda i: (0, i, 0)),
        pl.BlockSpec((_BLK, D), lambda i: (i, 0)),
        pl.BlockSpec((D, D), lambda i: (0, 0)),
        pl.BlockSpec((D, D), lambda i: (0, 0)),
        pl.BlockSpec((1, D), lambda i: (0, 0)),
    ],
    out_specs=pl.BlockSpec((_BLK, D), lambda i: (i, 0)),
    out_shape=jax.ShapeDtypeStruct((N_NODES, D), jnp.float32),
)


def kernel(x, edge_index, edge_attr, W_rel1, b_rel1, W_root1,
           W_rel2, b_rel2, W_root2):
    pad = E_PAD - N_EDGES
    src = jnp.concatenate([edge_index[0], jnp.zeros((pad,), jnp.int32)])
    dst = jnp.concatenate([edge_index[1], jnp.zeros((pad,), jnp.int32)])
    w = jnp.concatenate([edge_attr, jnp.zeros((pad,), jnp.float32)])
    wbits = lax.bitcast_convert_type(w, jnp.int32)
    packed = jnp.stack(
        [src.reshape(N_CHUNKS, CHUNK),
         dst.reshape(N_CHUNKS, CHUNK),
         wbits.reshape(N_CHUNKS, CHUNK)], axis=1)
    zeros = jnp.zeros((N_NODES, D), jnp.float32)

    p1 = _sc_agg(x, packed, zeros)
    h1 = _tc_combine(p1, x, W_rel1, W_root1, b_rel1.reshape(1, D))
    p2 = _sc_agg(h1, packed, zeros)
    h2 = _tc_combine(p2, h1, W_rel2, W_root2, b_rel2.reshape(1, D))
    return h2


# speedup vs baseline: 1.3349x; 1.3349x over previous
import functools

import jax
import jax.numpy as jnp
from jax import lax
from jax.experimental import pallas as pl
from jax.experimental.pallas import tpu as pltpu
from jax.experimental.pallas import tpu_sc as plsc

N_NODES = 10000
N_EDGES = 320000
D = 128

NC = 2
NS = 16
L = 16

CHUNK = 128
NCH = 80
N_CHUNKS = NC * NS * NCH
E_PAD = N_CHUNKS * CHUNK

ROWS_PER_TILE = 624
REM_BASE = ROWS_PER_TILE * NS
REM_ROWS = N_NODES - REM_BASE

_mesh = plsc.VectorSubcoreMesh(core_axis_name="c", subcore_axis_name="s")


@functools.partial(
    pl.kernel,
    out_type=jax.ShapeDtypeStruct((NC, N_NODES, D), jnp.float32),
    mesh=_mesh,
    compiler_params=pltpu.CompilerParams(needs_layout_passes=False),
    scratch_types=[
        pltpu.VMEM_SHARED((N_NODES, D), jnp.float32),
        pltpu.VMEM((CHUNK,), jnp.int32),
        pltpu.VMEM((CHUNK,), jnp.int32),
        pltpu.VMEM((CHUNK,), jnp.int32),
        pltpu.VMEM((CHUNK,), jnp.int32),
        pltpu.VMEM((CHUNK,), jnp.int32),
        pltpu.VMEM((CHUNK,), jnp.int32),
        pltpu.VMEM((CHUNK,), jnp.float32),
        pltpu.VMEM((CHUNK,), jnp.float32),
        pltpu.VMEM((CHUNK,), jnp.float32),
        pltpu.VMEM((CHUNK, D), jnp.float32),
        pltpu.VMEM((CHUNK, D), jnp.float32),
        pltpu.VMEM((CHUNK, D), jnp.float32),
        pltpu.SemaphoreType.DMA,
        pltpu.SemaphoreType.DMA,
        pltpu.SemaphoreType.DMA,
        pltpu.SemaphoreType.DMA,
        pltpu.SemaphoreType.DMA,
        pltpu.SemaphoreType.DMA,
        pltpu.SemaphoreType.DMA,
        pltpu.SemaphoreType.DMA,
        pltpu.SemaphoreType.DMA,
    ],
)
def _sc_agg(h_hbm, src_hbm, dst_hbm, w_hbm, zeros_hbm, out_hbm,
            acc, sv0, sv1, sv2, dv0, dv1, dv2, wv0, wv1, wv2,
            rb0, rb1, rb2,
            si0, si1, si2, sg0, sg1, sg2, ss0, ss1, ss2):
    cid = lax.axis_index("c")
    sid = lax.axis_index("s")
    wid = sid * NC + cid
    first = wid * NCH

    svs = (sv0, sv1, sv2)
    dvs = (dv0, dv1, dv2)
    wvs = (wv0, wv1, wv2)
    rbs = (rb0, rb1, rb2)
    sis = (si0, si1, si2)
    sgs = (sg0, sg1, sg2)
    sss = (ss0, ss1, ss2)

    pltpu.sync_copy(zeros_hbm.at[pl.ds(sid * ROWS_PER_TILE, ROWS_PER_TILE)],
                    acc.at[pl.ds(sid * ROWS_PER_TILE, ROWS_PER_TILE)])

    @pl.when(sid == NS - 1)
    def _zero_rem():
        pltpu.sync_copy(zeros_hbm.at[pl.ds(REM_BASE, REM_ROWS)],
                        acc.at[pl.ds(REM_BASE, REM_ROWS)])

    plsc.subcore_barrier()

    def idx_start(c, p):
        off = (first + c) * CHUNK
        pltpu.async_copy(src_hbm.at[pl.ds(off, CHUNK)], svs[p], sis[p])
        pltpu.async_copy(dst_hbm.at[pl.ds(off, CHUNK)], dvs[p], sis[p])
        pltpu.async_copy(w_hbm.at[pl.ds(off, CHUNK)], wvs[p], sis[p])

    def idx_wait(p):
        pltpu.make_async_copy(src_hbm.at[pl.ds(0, CHUNK)], svs[p], sis[p]).wait()
        pltpu.make_async_copy(dst_hbm.at[pl.ds(0, CHUNK)], dvs[p], sis[p]).wait()
        pltpu.make_async_copy(w_hbm.at[pl.ds(0, CHUNK)], wvs[p], sis[p]).wait()

    def gather_start(p):
        pltpu.async_copy(h_hbm.at[svs[p]], rbs[p], sgs[p])

    def gather_wait(p):
        pltpu.make_async_copy(h_hbm.at[svs[p]], rbs[p], sgs[p]).wait()

    def scatter_start(p):
        pltpu.async_copy(rbs[p], acc.at[dvs[p]], sss[p], add=True)

    def scatter_wait(p):
        pltpu.make_async_copy(rbs[p], acc.at[dvs[p]], sss[p]).wait()

    def scale(p):
        def body(e, c2):
            w16 = plsc.load_gather(wvs[p], [jnp.broadcast_to(e, (L,))])
            for j in range(D // L):
                sl = pl.ds(j * L, L)
                rbs[p][e, sl] = rbs[p][e, sl] * w16
            return c2

        lax.fori_loop(0, CHUNK, body, 0, unroll=2)

    def run_iter(i, p_cur, p_next, p_new, sw=True, di=True, dg=True):
        gather_wait(p_cur)
        if sw:
            scatter_wait(p_new)
        if di:
            idx_start(i, p_new)
        if dg:
            idx_wait(p_next)
            gather_start(p_next)
        scale(p_cur)
        scatter_start(p_cur)

    idx_start(0, 0)
    idx_start(1, 1)
    idx_wait(0)
    gather_start(0)

    run_iter(2, 0, 1, 2, sw=False)
    run_iter(3, 1, 2, 0)
    run_iter(4, 2, 0, 1)

    def block(j, carry):
        i = 5 + 3 * j
        run_iter(i, 0, 1, 2)
        run_iter(i + 1, 1, 2, 0)
        run_iter(i + 2, 2, 0, 1)
        return carry

    lax.fori_loop(0, (NCH - 5) // 3, block, 0)

    run_iter(NCH, 0, 1, 2, di=False)
    run_iter(NCH + 1, 1, 2, 0, di=False, dg=False)
    scatter_wait(1)

    plsc.subcore_barrier()

    pltpu.sync_copy(acc.at[pl.ds(sid * ROWS_PER_TILE, ROWS_PER_TILE)],
                    out_hbm.at[cid, pl.ds(sid * ROWS_PER_TILE, ROWS_PER_TILE)])

    @pl.when(sid == NS - 1)
    def _write_rem():
        pltpu.sync_copy(acc.at[pl.ds(REM_BASE, REM_ROWS)],
                        out_hbm.at[cid, pl.ds(REM_BASE, REM_ROWS)])


_BLK = 1000


def _tc_body(p_ref, h_ref, wrel_ref, wroot_ref, b_ref, o_ref):
    agg = p_ref[0] + p_ref[1]
    o_ref[...] = (
        jnp.dot(agg, wrel_ref[...], preferred_element_type=jnp.float32)
        + jnp.dot(h_ref[...], wroot_ref[...], preferred_element_type=jnp.float32)
        + b_ref[...]
    )


_tc_combine = pl.pallas_call(
    _tc_body,
    grid=(N_NODES // _BLK,),
    in_specs=[
        pl.BlockSpec((NC, _BLK, D), lambda i: (0, i, 0)),
        pl.BlockSpec((_BLK, D), lambda i: (i, 0)),
        pl.BlockSpec((D, D), lambda i: (0, 0)),
        pl.BlockSpec((D, D), lambda i: (0, 0)),
        pl.BlockSpec((1, D), lambda i: (0, 0)),
    ],
    out_specs=pl.BlockSpec((_BLK, D), lambda i: (i, 0)),
    out_shape=jax.ShapeDtypeStruct((N_NODES, D), jnp.float32),
)


def kernel(x, edge_index, edge_attr, W_rel1, b_rel1, W_root1,
           W_rel2, b_rel2, W_root2):
    pad = E_PAD - N_EDGES
    src = jnp.concatenate([edge_index[0], jnp.zeros((pad,), jnp.int32)])
    dst = jnp.concatenate([edge_index[1], jnp.zeros((pad,), jnp.int32)])
    w = jnp.concatenate([edge_attr, jnp.zeros((pad,), jnp.float32)])
    zeros = jnp.zeros((N_NODES, D), jnp.float32)

    p1 = _sc_agg(x, src, dst, w, zeros)
    h1 = _tc_combine(p1, x, W_rel1, W_root1, b_rel1.reshape(1, D))
    p2 = _sc_agg(h1, src, dst, w, zeros)
    h2 = _tc_combine(p2, h1, W_rel2, W_root2, b_rel2.reshape(1, D))
    return h2


# speedup vs baseline: 1.7411x; 1.3043x over previous
import functools

import jax
import jax.numpy as jnp
from jax import lax
from jax.experimental import pallas as pl
from jax.experimental.pallas import tpu as pltpu
from jax.experimental.pallas import tpu_sc as plsc

N_NODES = 10000
N_EDGES = 320000
D = 128

NC = 2
NS = 16
L = 16

CHUNK = 128
N_CHUNKS = N_EDGES // CHUNK
ROWS_PER_TILE = 624
REM_BASE = ROWS_PER_TILE * NS
REM_ROWS = N_NODES - REM_BASE

_mesh = plsc.VectorSubcoreMesh(core_axis_name="c", subcore_axis_name="s")


@functools.partial(
    pl.kernel,
    out_type=jax.ShapeDtypeStruct((NC, N_NODES, D), jnp.float32),
    mesh=_mesh,
    compiler_params=pltpu.CompilerParams(needs_layout_passes=False),
    scratch_types=[
        pltpu.VMEM_SHARED((N_NODES, D), jnp.float32),
        pltpu.VMEM((CHUNK,), jnp.int32),
        pltpu.VMEM((CHUNK,), jnp.int32),
        pltpu.VMEM((CHUNK,), jnp.float32),
        pltpu.VMEM((CHUNK, D), jnp.float32),
        pltpu.SemaphoreType.DMA,
    ],
)
def _sc_agg(h_hbm, src_hbm, dst_hbm, w_hbm, zeros_hbm, out_hbm,
            acc, src_v, dst_v, w_v, rows_v, sem):
    cid = lax.axis_index("c")
    sid = lax.axis_index("s")
    wid = sid * NC + cid

    pltpu.sync_copy(zeros_hbm.at[pl.ds(sid * ROWS_PER_TILE, ROWS_PER_TILE)],
                    acc.at[pl.ds(sid * ROWS_PER_TILE, ROWS_PER_TILE)])

    @pl.when(sid == NS - 1)
    def _zero_rem():
        pltpu.sync_copy(zeros_hbm.at[pl.ds(REM_BASE, REM_ROWS)],
                        acc.at[pl.ds(REM_BASE, REM_ROWS)])

    plsc.subcore_barrier()

    n_base = N_CHUNKS // (NC * NS)
    n_rem = N_CHUNKS % (NC * NS)
    my_n = jnp.where(wid < n_rem, n_base + 1, n_base)
    my_start = wid * n_base + jnp.minimum(wid, n_rem)

    def chunk_body(g, carry):
        base = (my_start + g) * CHUNK
        pltpu.sync_copy(src_hbm.at[pl.ds(base, CHUNK)], src_v)
        pltpu.sync_copy(dst_hbm.at[pl.ds(base, CHUNK)], dst_v)
        pltpu.sync_copy(w_hbm.at[pl.ds(base, CHUNK)], w_v)
        pltpu.async_copy(h_hbm.at[src_v], rows_v, sem).wait()

        def scale_body(e, c2):
            w16 = plsc.load_gather(w_v, [jnp.broadcast_to(e, (L,))])
            for j in range(D // L):
                sl = pl.ds(j * L, L)
                rows_v[e, sl] = rows_v[e, sl] * w16
            return c2

        lax.fori_loop(0, CHUNK, scale_body, 0, unroll=2)
        pltpu.sync_copy(rows_v, acc.at[dst_v], add=True)
        return carry

    lax.fori_loop(0, my_n, chunk_body, 0)
    plsc.subcore_barrier()

    pltpu.sync_copy(acc.at[pl.ds(sid * ROWS_PER_TILE, ROWS_PER_TILE)],
                    out_hbm.at[cid, pl.ds(sid * ROWS_PER_TILE, ROWS_PER_TILE)])

    @pl.when(sid == NS - 1)
    def _write_rem():
        pltpu.sync_copy(acc.at[pl.ds(REM_BASE, REM_ROWS)],
                        out_hbm.at[cid, pl.ds(REM_BASE, REM_ROWS)])


_BLK = 1000


def _tc_body(p_ref, h_ref, wrel_ref, wroot_ref, b_ref, o_ref):
    agg = p_ref[0] + p_ref[1]
    o_ref[...] = (
        jnp.dot(agg, wrel_ref[...], preferred_element_type=jnp.float32)
        + jnp.dot(h_ref[...], wroot_ref[...], preferred_element_type=jnp.float32)
        + b_ref[...]
    )


_tc_combine = pl.pallas_call(
    _tc_body,
    grid=(N_NODES // _BLK,),
    in_specs=[
        pl.BlockSpec((NC, _BLK, D), lambda i: (0, i, 0)),
        pl.BlockSpec((_BLK, D), lambda i: (i, 0)),
        pl.BlockSpec((D, D), lambda i: (0, 0)),
        pl.BlockSpec((D, D), lambda i: (0, 0)),
        pl.BlockSpec((1, D), lambda i: (0, 0)),
    ],
    out_specs=pl.BlockSpec((_BLK, D), lambda i: (i, 0)),
    out_shape=jax.ShapeDtypeStruct((N_NODES, D), jnp.float32),
)


def kernel(x, edge_index, edge_attr, W_rel1, b_rel1, W_root1,
           W_rel2, b_rel2, W_root2):
    src = edge_index[0]
    dst = edge_index[1]
    zeros = jnp.zeros((N_NODES, D), jnp.float32)

    p1 = _sc_agg(x, src, dst, edge_attr, zeros)
    h1 = _tc_combine(p1, x, W_rel1, W_root1, b_rel1.reshape(1, D))
    p2 = _sc_agg(h1, src, dst, edge_attr, zeros)
    h2 = _tc_combine(p2, h1, W_rel2, W_root2, b_rel2.reshape(1, D))
    return h2
